# baseline (speedup 1.0000x reference)
import functools

import jax
import jax.numpy as jnp
import numpy as np
from jax import lax
from jax.experimental import pallas as pl
from jax.experimental.pallas import tpu as pltpu
from jax.experimental.pallas import tpu_sc as plsc

N0 = 50000
N1 = 12500
H = 16
K = 15
EXT0 = 0.5
EXT1 = 1.0

_NW = 32
_CH = 128

_G = 4
_CPW = _NW * _CH * 2 * _G
_E0P = 819200
_M0 = _E0P // H
_E1P = 229376
_M1 = _E1P // H
_B0 = 1600
_B1 = 1792


def _kp(extent, seed):
    rng = np.random.RandomState(seed)
    kp = rng.uniform(-1.0, 1.0, size=(K, 3)).astype(np.float32)
    kp[0, :] = 0.0
    return kp * extent


_KP0 = _kp(EXT0, 7)
_KP1 = _kp(EXT1, 8)

_HK = H * K


def _np_consts():
    selc = np.zeros((H * 16, H * 3), np.float32)
    self_ = np.zeros((H * 16, H), np.float32)
    for h in range(H):
        for c in range(3):
            selc[h * 16 + c, c * H + h] = 1.0
        self_[h * 16 + 3, h] = 1.0
    tile4 = np.zeros((4, H * 3), np.float32)
    for h in range(H):
        for c in range(3):
            tile4[c, c * H + h] = 1.0
    rh = np.zeros((H, _HK), np.float32)
    for h in range(H):
        for k in range(K):
            rh[h, h * K + k] = 1.0
    return selc, self_, tile4, rh


_SELC, _SELF, _TILE4, _RH = _np_consts()


def _kpc(kp):
    m = np.zeros((4, _HK), np.float32)
    for h in range(H):
        for k in range(K):
            for c in range(3):
                m[c, h * K + k] = kp[k, c]
    return m


_KPC0 = _kpc(_KP0)
_KPC1 = _kpc(_KP1)


def _rkc(c_in):
    m = np.zeros((K, K * c_in), np.float32)
    for k in range(K):
        m[k, k * c_in:(k + 1) * c_in] = 1.0
    return m


def _rck(c_in):
    m = np.zeros((c_in, K * c_in), np.float32)
    for k in range(K):
        for c in range(c_in):
            m[c, k * c_in + c] = 1.0
    return m


_RK16, _RC16 = _rkc(16), _rck(16)
_RK32, _RC32 = _rkc(32), _rck(32)


def _lrelu(x):
    return jnp.where(x > 0, x, 0.1 * x)


def _dot(a, b):
    return jnp.dot(a, b, preferred_element_type=jnp.float32)


def _sc_gather(table, idx):
    V, D = table.shape
    E = idx.shape[0]
    Epad = -(-E // _CPW) * _CPW
    nch = Epad // (_NW * _CH)
    ntot = nch // (2 * _G)
    gr = _G * _CH
    idx2 = jnp.pad(idx.astype(jnp.int32), (0, Epad - E)).reshape(_NW, nch, _CH)
    mesh = plsc.VectorSubcoreMesh(core_axis_name="c", subcore_axis_name="s")

    @functools.partial(
        pl.kernel,
        mesh=mesh,
        compiler_params=pltpu.CompilerParams(use_tc_tiling_on_sc=False),
        out_type=jax.ShapeDtypeStruct((Epad, D), jnp.float32),
        scratch_types=[
            pltpu.VMEM((nch, _CH), jnp.int32),
            pltpu.VMEM((gr, D), jnp.float32),
            pltpu.SemaphoreType.DMA,
        ],
    )
    def gk(table_h, idx_h, out_h, idx_v, slab, sem):
        wid = lax.axis_index("s") * 2 + lax.axis_index("c")
        base = wid * nch * _CH
        pltpu.sync_copy(idx_h.at[wid], idx_v)

        def body(t, carry):
            j = t * _G
            for b in range(_G):
                pltpu.async_copy(table_h.at[idx_v.at[j + b]],
                                 slab.at[pl.ds(b * _CH, _CH)], sem)
            pltpu.make_async_copy(table_h.at[pl.ds(0, gr)], slab, sem).wait()
            pltpu.sync_copy(slab, out_h.at[pl.ds(base + j * _CH, gr)])
            return carry

        lax.fori_loop(0, nch // _G, body, 0)

    return gk(table, idx2)


def _infl_from(g, q4, selc, tile4, rh, kpc, inv_ext):
    rel = _dot(g, selc) - _dot(q4, tile4)
    dx = _dot(rel[:, 0:H], rh) - kpc[0:1, :]
    dy = _dot(rel[:, H:2 * H], rh) - kpc[1:2, :]
    dz = _dot(rel[:, 2 * H:3 * H], rh) - kpc[2:3, :]
    d = jnp.sqrt(dx * dx + dy * dy + dz * dz + 1e-12)
    return jnp.maximum(1.0 - d * inv_ext, 0.0)


def _t1_body(g_ref, q_ref, selc_ref, self_ref, tile4_ref,
             rh_ref, kpc_ref, ws2_ref, bs_ref, w1a_ref, b1a_ref,
             x_ref, h1_ref, infl_ref):
    infl = _infl_from(g_ref[...], q_ref[...], selc_ref[...], tile4_ref[...],
                      rh_ref[...], kpc_ref[...], 1.0 / EXT0)
    infl_ref[...] = infl
    f_nb = _dot(g_ref[...], self_ref[...])
    prod = infl * _dot(f_nb, rh_ref[...])
    x = _lrelu(_dot(prod, ws2_ref[...]) + bs_ref[...])
    x_ref[...] = x
    h1_ref[...] = _lrelu(_dot(x, w1a_ref[...]) + b1a_ref[...])


def _agg(infl, gnx, rk, rc, c_in):
    acc = None
    for h in range(H):
        ir = _dot(infl[:, K * h:K * (h + 1)], rk)
        nr = _dot(gnx[:, c_in * h:c_in * (h + 1)], rc)
        acc = ir * nr if acc is None else acc + ir * nr
    return acc


def _t2_body(infl_ref, gh1_ref, x_ref, rk_ref, rc_ref, wf_ref, b1c_ref,
             w1b_ref, b1b_ref, w1sc_ref, w2a_ref, b2a_ref,
             skip_ref, h2a_ref):
    acc = _agg(infl_ref[...], gh1_ref[...], rk_ref[...], rc_ref[...], 16)
    h = _lrelu(_dot(acc, wf_ref[...]) + b1c_ref[...])
    hb = _dot(h, w1b_ref[...]) + b1b_ref[...]
    skip0 = _lrelu(hb + _dot(x_ref[...], w1sc_ref[...]))
    skip_ref[...] = skip0
    h2a_ref[...] = _lrelu(_dot(skip0, w2a_ref[...]) + b2a_ref[...])


def _t3_body(g_ref, q_ref, gh2_ref, pmax_ref, selc_ref, tile4_ref,
             rh_ref, kpc_ref, rk_ref, rc_ref, wf_ref, b2c_ref,
             w2b_ref, b2b_ref, w2sc_ref, w3a_ref, b3a_ref,
             x1_ref, h3a_ref):
    infl = _infl_from(g_ref[...], q_ref[...], selc_ref[...], tile4_ref[...],
                      rh_ref[...], kpc_ref[...], 1.0 / EXT1)
    acc = _agg(infl, gh2_ref[...], rk_ref[...], rc_ref[...], 32)
    h = _lrelu(_dot(acc, wf_ref[...]) + b2c_ref[...])
    x1 = _lrelu(_dot(h, w2b_ref[...]) + b2b_ref[...]
                + _dot(pmax_ref[...], w2sc_ref[...]))
    x1_ref[...] = x1
    h3a_ref[...] = _lrelu(_dot(x1, w3a_ref[...]) + b3a_ref[...])


def _t4_body(g_ref, q_ref, g3_ref, x1_ref, selc_ref, tile4_ref,
             rh_ref, kpc_ref, rk_ref, rc_ref, wf_ref, bc_ref,
             wb_ref, bb_ref, wa_ref, ba_ref,
             x1o_ref, hna_ref, infl_ref):
    infl = _infl_from(g_ref[...], q_ref[...], selc_ref[...], tile4_ref[...],
                      rh_ref[...], kpc_ref[...], 1.0 / EXT1)
    infl_ref[...] = infl
    acc = _agg(infl, g3_ref[...], rk_ref[...], rc_ref[...], 32)
    h = _lrelu(_dot(acc, wf_ref[...]) + bc_ref[...])
    x1o = _lrelu(_dot(h, wb_ref[...]) + bb_ref[...] + x1_ref[...])
    x1o_ref[...] = x1o
    hna_ref[...] = _lrelu(_dot(x1o, wa_ref[...]) + ba_ref[...])


def _t5_body(infl_ref, g4_ref, x1_ref, rk_ref, rc_ref, wf_ref, bc_ref,
             wb_ref, bb_ref, xlo_ref, xhi_ref):
    acc = _agg(infl_ref[...], g4_ref[...], rk_ref[...], rc_ref[...], 32)
    h = _lrelu(_dot(acc, wf_ref[...]) + bc_ref[...])
    x1o = _lrelu(_dot(h, wb_ref[...]) + bb_ref[...] + x1_ref[...])
    xlo_ref[...] = x1o[:, :64]
    xhi_ref[...] = x1o[:, 64:]


def _t6_body(ulo_ref, uhi_ref, skip_ref, mf_ref, wlo_ref, whi_ref, wls_ref,
             bl_ref, desc_ref, score_ref):
    f = (_dot(ulo_ref[...], wlo_ref[...]) + _dot(uhi_ref[...], whi_ref[...])
         + _dot(skip_ref[...], wls_ref[...]) + bl_ref[...])
    m = mf_ref[...]
    nrm = jnp.sqrt(jnp.sum(f * f, axis=1, keepdims=True))
    desc_ref[...] = f / (nrm + 1e-12)
    z = f - m
    lm = jnp.maximum(z, 0.0) + jnp.log1p(jnp.exp(-jnp.abs(z)))
    dmax = jnp.max(f, axis=1, keepdims=True)
    ds = f / (1e-6 + dmax)
    score_ref[...] = jnp.max(lm * ds, axis=1, keepdims=True)


def _tfeat_body(ulo_ref, uhi_ref, skip_ref, wlo_ref, whi_ref, wls_ref,
                bl_ref, feat_ref):
    feat_ref[...] = (_dot(ulo_ref[...], wlo_ref[...])
                     + _dot(uhi_ref[...], whi_ref[...])
                     + _dot(skip_ref[...], wls_ref[...]) + bl_ref[...])


def _const_spec(a):
    return pl.BlockSpec(a.shape, lambda i: tuple(0 for _ in a.shape))


def _row_spec(b, d):
    return pl.BlockSpec((b, d), lambda i: (i, 0))


def _call(body, grid, b, ins, row_ins, out_ds):
    in_specs = [_row_spec(b, a.shape[1]) for a in row_ins] + [_const_spec(a) for a in ins]
    return pl.pallas_call(
        body,
        grid=(grid,),
        in_specs=in_specs,
        out_specs=[_row_spec(b, d) for d in out_ds],
        out_shape=[jax.ShapeDtypeStruct((grid * b, d), jnp.float32) for d in out_ds],
    )(*row_ins, *ins)


def kernel(points_0, points_1, neighbors_0, neighbors_1, pools_0, upsamples_0, features, stack_lengths_0, stack_lengths_1, params):
    p = params
    f32 = jnp.float32

    pack0 = jnp.zeros((N0, 16), f32).at[:, :3].set(points_0).at[:, 3].set(features[:, 0])
    pack1 = jnp.zeros((N1, 16), f32).at[:, :3].set(points_1)
    q4_0 = jnp.pad(points_0, ((0, _M0 - N0), (0, 1)))
    q4_1 = jnp.pad(points_1, ((0, _M1 - N1), (0, 1)))
    e0 = neighbors_0.reshape(-1)
    e1 = neighbors_1.reshape(-1)
    ep = pools_0.reshape(-1)

    consts0 = [jnp.asarray(a) for a in (_SELC, _SELF, _TILE4, _RH, _KPC0)]
    selc, self_, tile4, rh, kpc0 = consts0
    kpc1 = jnp.asarray(_KPC1)
    rk16, rc16 = jnp.asarray(_RK16), jnp.asarray(_RC16)
    rk32, rc32 = jnp.asarray(_RK32), jnp.asarray(_RC32)

    ws2 = jnp.tile(p['W_s'][:, 0, :], (H, 1))
    row = lambda v: v.reshape(1, -1)

    g0 = _sc_gather(pack0, e0).reshape(_M0, 256)
    gpp = _sc_gather(pack0, ep).reshape(_M1, 256)
    gp1 = _sc_gather(pack1, e1).reshape(_M1, 256)

    x, h1, infl0 = _call(
        _t1_body, _M0 // _B0, _B0,
        [selc, self_, tile4, rh, kpc0, ws2, row(p['b_s']),
         p['W1a'], row(p['b1a'])],
        [g0, q4_0], [32, 16, _HK])

    gh1 = _sc_gather(h1, e0).reshape(_M0, 256)
    skip0, h2a = _call(
        _t2_body, _M0 // _B0, _B0,
        [rk16, rc16, p['W1c'].reshape(K * 16, 16), row(p['b1c']), p['W1b'],
         row(p['b1b']), p['W1sc'], p['W2a'], row(p['b2a'])],
        [infl0, gh1, x], [64, 32])

    gh2 = _sc_gather(h2a, ep).reshape(_M1, 512)
    pmax = jnp.max(_sc_gather(skip0, ep).reshape(_M1, H, 64), axis=1)
    x1, h3a = _call(
        _t3_body, _M1 // _B1, _B1,
        [selc, tile4, rh, kpc1, rk32, rc32,
         p['W2c'].reshape(K * 32, 32), row(p['b2c']), p['W2b'], row(p['b2b']),
         p['W2sc'], p['W3a'], row(p['b3a'])],
        [gpp, q4_1, gh2, pmax], [128, 32])

    g3 = _sc_gather(h3a, e1).reshape(_M1, 512)
    x1b, h4a, infl1 = _call(
        _t4_body, _M1 // _B1, _B1,
        [selc, tile4, rh, kpc1, rk32, rc32,
         p['W3c'].reshape(K * 32, 32), row(p['b3c']), p['W3b'], row(p['b3b']),
         p['W4a'], row(p['b4a'])],
        [gp1, q4_1, g3, x1], [128, 32, _HK])

    g4 = _sc_gather(h4a, e1).reshape(_M1, 512)
    x1lo, x1hi = _call(
        _t5_body, _M1 // _B1, _B1,
        [rk32, rc32, p['W4c'].reshape(K * 32, 32), row(p['b4c']), p['W4b'],
         row(p['b4b'])],
        [infl1, g4, x1b], [64, 64])

    upidx = jnp.pad(upsamples_0[:, 0], (0, _M0 - N0))
    ulo = _sc_gather(x1lo, upidx)[:_M0]
    uhi = _sc_gather(x1hi, upidx)[:_M0]
    wlo, whi, wls = p['Wl'][:64], p['Wl'][64:128], p['Wl'][128:]
    (feat,) = _call(
        _tfeat_body, _M0 // _B0, _B0,
        [wlo, whi, wls, row(p['bl'])],
        [ulo, uhi, skip0], [32])
    mf = jnp.mean(_sc_gather(feat, e0).reshape(_M0, H, 32), axis=1)

    desc, scores = _call(
        _t6_body, _M0 // _B0, _B0,
        [wlo, whi, wls, row(p['bl'])],
        [ulo, uhi, skip0, mf], [32, 1])

    return desc[:N0], scores[:N0]


# speedup vs baseline: 1.3183x; 1.3183x over previous
import functools

import jax
import jax.numpy as jnp
import numpy as np
from jax import lax
from jax.experimental import pallas as pl
from jax.experimental.pallas import tpu as pltpu
from jax.experimental.pallas import tpu_sc as plsc

N0 = 50000
N1 = 12500
H = 16
K = 15
EXT0 = 0.5
EXT1 = 1.0

_NW = 32
_CH = 512

_CPW = _NW * _CH
_E0P = 802816
_M0 = _E0P // H
_E1P = 212992
_M1 = _E1P // H
_B0 = 1568
_B1 = 1664


def _kp(extent, seed):
    rng = np.random.RandomState(seed)
    kp = rng.uniform(-1.0, 1.0, size=(K, 3)).astype(np.float32)
    kp[0, :] = 0.0
    return kp * extent


_KP0 = _kp(EXT0, 7)
_KP1 = _kp(EXT1, 8)

_HK = H * K


def _np_consts():
    selc = np.zeros((H * 16, H * 3), np.float32)
    self_ = np.zeros((H * 16, H), np.float32)
    for h in range(H):
        for c in range(3):
            selc[h * 16 + c, c * H + h] = 1.0
        self_[h * 16 + 3, h] = 1.0
    tile4 = np.zeros((4, H * 3), np.float32)
    for h in range(H):
        for c in range(3):
            tile4[c, c * H + h] = 1.0
    rh = np.zeros((H, _HK), np.float32)
    for h in range(H):
        for k in range(K):
            rh[h, h * K + k] = 1.0
    return selc, self_, tile4, rh


_SELC, _SELF, _TILE4, _RH = _np_consts()


def _kpc(kp):
    m = np.zeros((4, _HK), np.float32)
    for h in range(H):
        for k in range(K):
            for c in range(3):
                m[c, h * K + k] = kp[k, c]
    return m


_KPC0 = _kpc(_KP0)
_KPC1 = _kpc(_KP1)


def _rkc(c_in):
    m = np.zeros((K, K * c_in), np.float32)
    for k in range(K):
        m[k, k * c_in:(k + 1) * c_in] = 1.0
    return m


def _rck(c_in):
    m = np.zeros((c_in, K * c_in), np.float32)
    for k in range(K):
        for c in range(c_in):
            m[c, k * c_in + c] = 1.0
    return m


_RK16, _RC16 = _rkc(16), _rck(16)
_RK32, _RC32 = _rkc(32), _rck(32)


def _lrelu(x):
    return jnp.where(x > 0, x, 0.1 * x)


def _dot(a, b):
    return jnp.dot(a, b, preferred_element_type=jnp.float32)


def _sc_gather(table, idx):
    V, D = table.shape
    E = idx.shape[0]
    Epad = -(-E // _CPW) * _CPW
    nch = Epad // (_NW * _CH)
    idx2 = jnp.pad(idx.astype(jnp.int32), (0, Epad - E)).reshape(_NW, nch, _CH)
    mesh = plsc.VectorSubcoreMesh(core_axis_name="c", subcore_axis_name="s")

    @functools.partial(
        pl.kernel,
        mesh=mesh,
        compiler_params=pltpu.CompilerParams(use_tc_tiling_on_sc=False),
        out_type=jax.ShapeDtypeStruct((Epad, D), jnp.float32),
        scratch_types=[
            pltpu.VMEM((nch, _CH), jnp.int32),
            pltpu.VMEM((_CH, D), jnp.float32),
            pltpu.SemaphoreType.DMA,
        ],
    )
    def gk(table_h, idx_h, out_h, idx_v, rows_v, sem):
        wid = lax.axis_index("s") * 2 + lax.axis_index("c")
        pltpu.sync_copy(idx_h.at[wid], idx_v)

        def body(j, carry):
            pltpu.async_copy(table_h.at[idx_v.at[j]], rows_v, sem).wait()
            pltpu.sync_copy(rows_v, out_h.at[pl.ds((wid * nch + j) * _CH, _CH)])
            return carry

        lax.fori_loop(0, nch, body, 0)

    return gk(table, idx2)


def _infl_from(g, q4, selc, tile4, rh, kpc, inv_ext):
    rel = _dot(g, selc) - _dot(q4, tile4)
    dx = _dot(rel[:, 0:H], rh) - kpc[0:1, :]
    dy = _dot(rel[:, H:2 * H], rh) - kpc[1:2, :]
    dz = _dot(rel[:, 2 * H:3 * H], rh) - kpc[2:3, :]
    d = jnp.sqrt(dx * dx + dy * dy + dz * dz + 1e-12)
    return jnp.maximum(1.0 - d * inv_ext, 0.0)


def _t1_body(g_ref, q_ref, selc_ref, self_ref, tile4_ref,
             rh_ref, kpc_ref, ws2_ref, bs_ref, w1a_ref, b1a_ref,
             x_ref, h1_ref, infl_ref):
    infl = _infl_from(g_ref[...], q_ref[...], selc_ref[...], tile4_ref[...],
                      rh_ref[...], kpc_ref[...], 1.0 / EXT0)
    infl_ref[...] = infl
    f_nb = _dot(g_ref[...], self_ref[...])
    prod = infl * _dot(f_nb, rh_ref[...])
    x = _lrelu(_dot(prod, ws2_ref[...]) + bs_ref[...])
    x_ref[...] = x
    h1_ref[...] = _lrelu(_dot(x, w1a_ref[...]) + b1a_ref[...])


def _agg(infl, gnx, rk, rc, c_in):
    acc = None
    for h in range(H):
        ir = _dot(infl[:, K * h:K * (h + 1)], rk)
        nr = _dot(gnx[:, c_in * h:c_in * (h + 1)], rc)
        acc = ir * nr if acc is None else acc + ir * nr
    return acc


def _t2_body(infl_ref, gh1_ref, x_ref, rk_ref, rc_ref, wf_ref, b1c_ref,
             w1b_ref, b1b_ref, w1sc_ref, w2a_ref, b2a_ref,
             skip_ref, h2a_ref):
    acc = _agg(infl_ref[...], gh1_ref[...], rk_ref[...], rc_ref[...], 16)
    h = _lrelu(_dot(acc, wf_ref[...]) + b1c_ref[...])
    hb = _dot(h, w1b_ref[...]) + b1b_ref[...]
    skip0 = _lrelu(hb + _dot(x_ref[...], w1sc_ref[...]))
    skip_ref[...] = skip0
    h2a_ref[...] = _lrelu(_dot(skip0, w2a_ref[...]) + b2a_ref[...])


def _t3_body(g_ref, q_ref, gh2_ref, pmax_ref, selc_ref, tile4_ref,
             rh_ref, kpc_ref, rk_ref, rc_ref, wf_ref, b2c_ref,
             w2b_ref, b2b_ref, w2sc_ref, w3a_ref, b3a_ref,
             x1_ref, h3a_ref):
    infl = _infl_from(g_ref[...], q_ref[...], selc_ref[...], tile4_ref[...],
                      rh_ref[...], kpc_ref[...], 1.0 / EXT1)
    acc = _agg(infl, gh2_ref[...], rk_ref[...], rc_ref[...], 32)
    h = _lrelu(_dot(acc, wf_ref[...]) + b2c_ref[...])
    x1 = _lrelu(_dot(h, w2b_ref[...]) + b2b_ref[...]
                + _dot(pmax_ref[...], w2sc_ref[...]))
    x1_ref[...] = x1
    h3a_ref[...] = _lrelu(_dot(x1, w3a_ref[...]) + b3a_ref[...])


def _t4_body(g_ref, q_ref, g3_ref, x1_ref, selc_ref, tile4_ref,
             rh_ref, kpc_ref, rk_ref, rc_ref, wf_ref, bc_ref,
             wb_ref, bb_ref, wa_ref, ba_ref,
             x1o_ref, hna_ref, infl_ref):
    infl = _infl_from(g_ref[...], q_ref[...], selc_ref[...], tile4_ref[...],
                      rh_ref[...], kpc_ref[...], 1.0 / EXT1)
    infl_ref[...] = infl
    acc = _agg(infl, g3_ref[...], rk_ref[...], rc_ref[...], 32)
    h = _lrelu(_dot(acc, wf_ref[...]) + bc_ref[...])
    x1o = _lrelu(_dot(h, wb_ref[...]) + bb_ref[...] + x1_ref[...])
    x1o_ref[...] = x1o
    hna_ref[...] = _lrelu(_dot(x1o, wa_ref[...]) + ba_ref[...])


def _t5_body(infl_ref, g4_ref, x1_ref, rk_ref, rc_ref, wf_ref, bc_ref,
             wb_ref, bb_ref, xlo_ref, xhi_ref):
    acc = _agg(infl_ref[...], g4_ref[...], rk_ref[...], rc_ref[...], 32)
    h = _lrelu(_dot(acc, wf_ref[...]) + bc_ref[...])
    x1o = _lrelu(_dot(h, wb_ref[...]) + bb_ref[...] + x1_ref[...])
    xlo_ref[...] = x1o[:, :64]
    xhi_ref[...] = x1o[:, 64:]


def _t6_body(ulo_ref, uhi_ref, skip_ref, mf_ref, wlo_ref, whi_ref, wls_ref,
             bl_ref, desc_ref, score_ref):
    f = (_dot(ulo_ref[...], wlo_ref[...]) + _dot(uhi_ref[...], whi_ref[...])
         + _dot(skip_ref[...], wls_ref[...]) + bl_ref[...])
    m = mf_ref[...]
    nrm = jnp.sqrt(jnp.sum(f * f, axis=1, keepdims=True))
    desc_ref[...] = f / (nrm + 1e-12)
    z = f - m
    lm = jnp.maximum(z, 0.0) + jnp.log1p(jnp.exp(-jnp.abs(z)))
    dmax = jnp.max(f, axis=1, keepdims=True)
    ds = f / (1e-6 + dmax)
    score_ref[...] = jnp.max(lm * ds, axis=1, keepdims=True)


def _tfeat_body(ulo_ref, uhi_ref, skip_ref, wlo_ref, whi_ref, wls_ref,
                bl_ref, feat_ref):
    feat_ref[...] = (_dot(ulo_ref[...], wlo_ref[...])
                     + _dot(uhi_ref[...], whi_ref[...])
                     + _dot(skip_ref[...], wls_ref[...]) + bl_ref[...])


def _const_spec(a):
    return pl.BlockSpec(a.shape, lambda i: tuple(0 for _ in a.shape))


def _row_spec(b, d):
    return pl.BlockSpec((b, d), lambda i: (i, 0))


def _call(body, grid, b, ins, row_ins, out_ds):
    in_specs = [_row_spec(b, a.shape[1]) for a in row_ins] + [_const_spec(a) for a in ins]
    return pl.pallas_call(
        body,
        grid=(grid,),
        in_specs=in_specs,
        out_specs=[_row_spec(b, d) for d in out_ds],
        out_shape=[jax.ShapeDtypeStruct((grid * b, d), jnp.float32) for d in out_ds],
    )(*row_ins, *ins)


def kernel(points_0, points_1, neighbors_0, neighbors_1, pools_0, upsamples_0, features, stack_lengths_0, stack_lengths_1, params):
    p = params
    f32 = jnp.float32

    pack0 = jnp.zeros((N0, 16), f32).at[:, :3].set(points_0).at[:, 3].set(features[:, 0])
    pack1 = jnp.zeros((N1, 16), f32).at[:, :3].set(points_1)
    q4_0 = jnp.pad(points_0, ((0, _M0 - N0), (0, 1)))
    q4_1 = jnp.pad(points_1, ((0, _M1 - N1), (0, 1)))
    e0 = neighbors_0.reshape(-1)
    e1 = neighbors_1.reshape(-1)
    ep = pools_0.reshape(-1)

    consts0 = [jnp.asarray(a) for a in (_SELC, _SELF, _TILE4, _RH, _KPC0)]
    selc, self_, tile4, rh, kpc0 = consts0
    kpc1 = jnp.asarray(_KPC1)
    rk16, rc16 = jnp.asarray(_RK16), jnp.asarray(_RC16)
    rk32, rc32 = jnp.asarray(_RK32), jnp.asarray(_RC32)

    ws2 = jnp.tile(p['W_s'][:, 0, :], (H, 1))
    row = lambda v: v.reshape(1, -1)

    g0 = _sc_gather(pack0, e0).reshape(_M0, 256)
    gpp = _sc_gather(pack0, ep).reshape(_M1, 256)
    gp1 = _sc_gather(pack1, e1).reshape(_M1, 256)

    x, h1, infl0 = _call(
        _t1_body, _M0 // _B0, _B0,
        [selc, self_, tile4, rh, kpc0, ws2, row(p['b_s']),
         p['W1a'], row(p['b1a'])],
        [g0, q4_0], [32, 16, _HK])

    gh1 = _sc_gather(h1, e0).reshape(_M0, 256)
    skip0, h2a = _call(
        _t2_body, _M0 // _B0, _B0,
        [rk16, rc16, p['W1c'].reshape(K * 16, 16), row(p['b1c']), p['W1b'],
         row(p['b1b']), p['W1sc'], p['W2a'], row(p['b2a'])],
        [infl0, gh1, x], [64, 32])

    gh2 = _sc_gather(h2a, ep).reshape(_M1, 512)
    pmax = jnp.max(_sc_gather(skip0, ep).reshape(_M1, H, 64), axis=1)
    x1, h3a = _call(
        _t3_body, _M1 // _B1, _B1,
        [selc, tile4, rh, kpc1, rk32, rc32,
         p['W2c'].reshape(K * 32, 32), row(p['b2c']), p['W2b'], row(p['b2b']),
         p['W2sc'], p['W3a'], row(p['b3a'])],
        [gpp, q4_1, gh2, pmax], [128, 32])

    g3 = _sc_gather(h3a, e1).reshape(_M1, 512)
    x1b, h4a, infl1 = _call(
        _t4_body, _M1 // _B1, _B1,
        [selc, tile4, rh, kpc1, rk32, rc32,
         p['W3c'].reshape(K * 32, 32), row(p['b3c']), p['W3b'], row(p['b3b']),
         p['W4a'], row(p['b4a'])],
        [gp1, q4_1, g3, x1], [128, 32, _HK])

    g4 = _sc_gather(h4a, e1).reshape(_M1, 512)
    x1lo, x1hi = _call(
        _t5_body, _M1 // _B1, _B1,
        [rk32, rc32, p['W4c'].reshape(K * 32, 32), row(p['b4c']), p['W4b'],
         row(p['b4b'])],
        [infl1, g4, x1b], [64, 64])

    upidx = jnp.pad(upsamples_0[:, 0], (0, _M0 - N0))
    ulo = _sc_gather(x1lo, upidx)[:_M0]
    uhi = _sc_gather(x1hi, upidx)[:_M0]
    wlo, whi, wls = p['Wl'][:64], p['Wl'][64:128], p['Wl'][128:]
    (feat,) = _call(
        _tfeat_body, _M0 // _B0, _B0,
        [wlo, whi, wls, row(p['bl'])],
        [ulo, uhi, skip0], [32])
    mf = jnp.mean(_sc_gather(feat, e0).reshape(_M0, H, 32), axis=1)

    desc, scores = _call(
        _t6_body, _M0 // _B0, _B0,
        [wlo, whi, wls, row(p['bl'])],
        [ulo, uhi, skip0, mf], [32, 1])

    return desc[:N0], scores[:N0]


# speedup vs baseline: 1.7126x; 1.2991x over previous
import functools

import jax
import jax.numpy as jnp
import numpy as np
from jax import lax
from jax.experimental import pallas as pl
from jax.experimental.pallas import tpu as pltpu
from jax.experimental.pallas import tpu_sc as plsc

N0 = 50000
N1 = 12500
H = 16
K = 15
EXT0 = 0.5
EXT1 = 1.0

_NW = 32
_CH = 128

_CPW = _NW * _CH
_E0P = 802816
_M0 = _E0P // H
_E1P = 200704
_M1 = _E1P // H
_B0 = 1568
_B1 = 1568


def _kp(extent, seed):
    rng = np.random.RandomState(seed)
    kp = rng.uniform(-1.0, 1.0, size=(K, 3)).astype(np.float32)
    kp[0, :] = 0.0
    return kp * extent


_KP0 = _kp(EXT0, 7)
_KP1 = _kp(EXT1, 8)

_HK = H * K


def _np_consts():
    selc = np.zeros((H * 16, H * 3), np.float32)
    self_ = np.zeros((H * 16, H), np.float32)
    for h in range(H):
        for c in range(3):
            selc[h * 16 + c, c * H + h] = 1.0
        self_[h * 16 + 3, h] = 1.0
    tile4 = np.zeros((4, H * 3), np.float32)
    for h in range(H):
        for c in range(3):
            tile4[c, c * H + h] = 1.0
    rh = np.zeros((H, _HK), np.float32)
    for h in range(H):
        for k in range(K):
            rh[h, h * K + k] = 1.0
    return selc, self_, tile4, rh


_SELC, _SELF, _TILE4, _RH = _np_consts()


def _kpc(kp):
    m = np.zeros((4, _HK), np.float32)
    for h in range(H):
        for k in range(K):
            for c in range(3):
                m[c, h * K + k] = kp[k, c]
    return m


_KPC0 = _kpc(_KP0)
_KPC1 = _kpc(_KP1)


def _rkc(c_in):
    m = np.zeros((K, K * c_in), np.float32)
    for k in range(K):
        m[k, k * c_in:(k + 1) * c_in] = 1.0
    return m


def _rck(c_in):
    m = np.zeros((c_in, K * c_in), np.float32)
    for k in range(K):
        for c in range(c_in):
            m[c, k * c_in + c] = 1.0
    return m


_RK16, _RC16 = _rkc(16), _rck(16)
_RK32, _RC32 = _rkc(32), _rck(32)


def _lrelu(x):
    return jnp.where(x > 0, x, 0.1 * x)


def _dot(a, b):
    return jnp.dot(a, b, preferred_element_type=jnp.float32)


def _sc_gather(table, idx):
    V, D = table.shape
    E = idx.shape[0]
    Epad = -(-E // _CPW) * _CPW
    nch = Epad // (_NW * _CH)
    idx2 = jnp.pad(idx.astype(jnp.int32), (0, Epad - E)).reshape(_NW, nch, _CH)
    mesh = plsc.VectorSubcoreMesh(core_axis_name="c", subcore_axis_name="s")

    @functools.partial(
        pl.kernel,
        mesh=mesh,
        compiler_params=pltpu.CompilerParams(use_tc_tiling_on_sc=False),
        out_type=jax.ShapeDtypeStruct((Epad, D), jnp.float32),
        scratch_types=[
            pltpu.VMEM((nch, _CH), jnp.int32),
            pltpu.VMEM((_CH, D), jnp.float32),
            pltpu.SemaphoreType.DMA,
        ],
    )
    def gk(table_h, idx_h, out_h, idx_v, rows_v, sem):
        wid = lax.axis_index("s") * 2 + lax.axis_index("c")
        pltpu.sync_copy(idx_h.at[wid], idx_v)

        def body(j, carry):
            pltpu.async_copy(table_h.at[idx_v.at[j]], rows_v, sem).wait()
            pltpu.sync_copy(rows_v, out_h.at[pl.ds((wid * nch + j) * _CH, _CH)])
            return carry

        lax.fori_loop(0, nch, body, 0)

    return gk(table, idx2)


def _infl_from(g, q4, selc, tile4, rh, kpc, inv_ext):
    rel = _dot(g, selc) - _dot(q4, tile4)
    dx = _dot(rel[:, 0:H], rh) - kpc[0:1, :]
    dy = _dot(rel[:, H:2 * H], rh) - kpc[1:2, :]
    dz = _dot(rel[:, 2 * H:3 * H], rh) - kpc[2:3, :]
    d = jnp.sqrt(dx * dx + dy * dy + dz * dz + 1e-12)
    return jnp.maximum(1.0 - d * inv_ext, 0.0)


def _t1_body(g_ref, q_ref, selc_ref, self_ref, tile4_ref,
             rh_ref, kpc_ref, ws2_ref, bs_ref, w1a_ref, b1a_ref,
             x_ref, h1_ref, infl_ref):
    infl = _infl_from(g_ref[...], q_ref[...], selc_ref[...], tile4_ref[...],
                      rh_ref[...], kpc_ref[...], 1.0 / EXT0)
    infl_ref[...] = infl
    f_nb = _dot(g_ref[...], self_ref[...])
    prod = infl * _dot(f_nb, rh_ref[...])
    x = _lrelu(_dot(prod, ws2_ref[...]) + bs_ref[...])
    x_ref[...] = x
    h1_ref[...] = _lrelu(_dot(x, w1a_ref[...]) + b1a_ref[...])


def _agg(infl, gnx, rk, rc, c_in):
    acc = None
    for h in range(H):
        ir = _dot(infl[:, K * h:K * (h + 1)], rk)
        nr = _dot(gnx[:, c_in * h:c_in * (h + 1)], rc)
        acc = ir * nr if acc is None else acc + ir * nr
    return acc


def _t2_body(infl_ref, gh1_ref, x_ref, rk_ref, rc_ref, wf_ref, b1c_ref,
             w1b_ref, b1b_ref, w1sc_ref, w2a_ref, b2a_ref,
             skip_ref, h2a_ref):
    acc = _agg(infl_ref[...], gh1_ref[...], rk_ref[...], rc_ref[...], 16)
    h = _lrelu(_dot(acc, wf_ref[...]) + b1c_ref[...])
    hb = _dot(h, w1b_ref[...]) + b1b_ref[...]
    skip0 = _lrelu(hb + _dot(x_ref[...], w1sc_ref[...]))
    skip_ref[...] = skip0
    h2a_ref[...] = _lrelu(_dot(skip0, w2a_ref[...]) + b2a_ref[...])


def _t3_body(g_ref, q_ref, gh2_ref, pmax_ref, selc_ref, tile4_ref,
             rh_ref, kpc_ref, rk_ref, rc_ref, wf_ref, b2c_ref,
             w2b_ref, b2b_ref, w2sc_ref, w3a_ref, b3a_ref,
             x1_ref, h3a_ref):
    infl = _infl_from(g_ref[...], q_ref[...], selc_ref[...], tile4_ref[...],
                      rh_ref[...], kpc_ref[...], 1.0 / EXT1)
    acc = _agg(infl, gh2_ref[...], rk_ref[...], rc_ref[...], 32)
    h = _lrelu(_dot(acc, wf_ref[...]) + b2c_ref[...])
    x1 = _lrelu(_dot(h, w2b_ref[...]) + b2b_ref[...]
                + _dot(pmax_ref[...], w2sc_ref[...]))
    x1_ref[...] = x1
    h3a_ref[...] = _lrelu(_dot(x1, w3a_ref[...]) + b3a_ref[...])


def _t4_body(g_ref, q_ref, g3_ref, x1_ref, selc_ref, tile4_ref,
             rh_ref, kpc_ref, rk_ref, rc_ref, wf_ref, bc_ref,
             wb_ref, bb_ref, wa_ref, ba_ref,
             x1o_ref, hna_ref, infl_ref):
    infl = _infl_from(g_ref[...], q_ref[...], selc_ref[...], tile4_ref[...],
                      rh_ref[...], kpc_ref[...], 1.0 / EXT1)
    infl_ref[...] = infl
    acc = _agg(infl, g3_ref[...], rk_ref[...], rc_ref[...], 32)
    h = _lrelu(_dot(acc, wf_ref[...]) + bc_ref[...])
    x1o = _lrelu(_dot(h, wb_ref[...]) + bb_ref[...] + x1_ref[...])
    x1o_ref[...] = x1o
    hna_ref[...] = _lrelu(_dot(x1o, wa_ref[...]) + ba_ref[...])


def _t5_body(infl_ref, g4_ref, x1_ref, rk_ref, rc_ref, wf_ref, bc_ref,
             wb_ref, bb_ref, x1o_ref):
    acc = _agg(infl_ref[...], g4_ref[...], rk_ref[...], rc_ref[...], 32)
    h = _lrelu(_dot(acc, wf_ref[...]) + bc_ref[...])
    x1o_ref[...] = _lrelu(_dot(h, wb_ref[...]) + bb_ref[...] + x1_ref[...])


def _t6_body(up_ref, skip_ref, mf_ref, wlu_ref, wls_ref,
             bl_ref, desc_ref, score_ref):
    f = (_dot(up_ref[...], wlu_ref[...])
         + _dot(skip_ref[...], wls_ref[...]) + bl_ref[...])
    m = mf_ref[...]
    nrm = jnp.sqrt(jnp.sum(f * f, axis=1, keepdims=True))
    desc_ref[...] = f / (nrm + 1e-12)
    z = f - m
    lm = jnp.maximum(z, 0.0) + jnp.log1p(jnp.exp(-jnp.abs(z)))
    dmax = jnp.max(f, axis=1, keepdims=True)
    ds = f / (1e-6 + dmax)
    score_ref[...] = jnp.max(lm * ds, axis=1, keepdims=True)


def _tfeat_body(up_ref, skip_ref, wlu_ref, wls_ref, bl_ref, feat_ref):
    feat_ref[...] = (_dot(up_ref[...], wlu_ref[...])
                     + _dot(skip_ref[...], wls_ref[...]) + bl_ref[...])


def _const_spec(a):
    return pl.BlockSpec(a.shape, lambda i: tuple(0 for _ in a.shape))


def _row_spec(b, d):
    return pl.BlockSpec((b, d), lambda i: (i, 0))


def _call(body, grid, b, ins, row_ins, out_ds):
    in_specs = [_row_spec(b, a.shape[1]) for a in row_ins] + [_const_spec(a) for a in ins]
    return pl.pallas_call(
        body,
        grid=(grid,),
        in_specs=in_specs,
        out_specs=[_row_spec(b, d) for d in out_ds],
        out_shape=[jax.ShapeDtypeStruct((grid * b, d), jnp.float32) for d in out_ds],
    )(*row_ins, *ins)


def kernel(points_0, points_1, neighbors_0, neighbors_1, pools_0, upsamples_0, features, stack_lengths_0, stack_lengths_1, params):
    p = params
    f32 = jnp.float32

    pack0 = jnp.zeros((N0, 16), f32).at[:, :3].set(points_0).at[:, 3].set(features[:, 0])
    pack1 = jnp.zeros((N1, 16), f32).at[:, :3].set(points_1)
    q4_0 = jnp.pad(points_0, ((0, _M0 - N0), (0, 1)))
    q4_1 = jnp.pad(points_1, ((0, _M1 - N1), (0, 1)))
    e0 = neighbors_0.reshape(-1)
    e1 = neighbors_1.reshape(-1)
    ep = pools_0.reshape(-1)

    consts0 = [jnp.asarray(a) for a in (_SELC, _SELF, _TILE4, _RH, _KPC0)]
    selc, self_, tile4, rh, kpc0 = consts0
    kpc1 = jnp.asarray(_KPC1)
    rk16, rc16 = jnp.asarray(_RK16), jnp.asarray(_RC16)
    rk32, rc32 = jnp.asarray(_RK32), jnp.asarray(_RC32)

    ws2 = jnp.tile(p['W_s'][:, 0, :], (H, 1))
    row = lambda v: v.reshape(1, -1)

    g0 = _sc_gather(pack0, e0).reshape(_M0, 256)
    gpp = _sc_gather(pack0, ep).reshape(_M1, 256)
    gp1 = _sc_gather(pack1, e1).reshape(_M1, 256)

    x, h1, infl0 = _call(
        _t1_body, _M0 // _B0, _B0,
        [selc, self_, tile4, rh, kpc0, ws2, row(p['b_s']),
         p['W1a'], row(p['b1a'])],
        [g0, q4_0], [32, 16, _HK])

    gh1 = _sc_gather(h1, e0).reshape(_M0, 256)
    skip0, h2a = _call(
        _t2_body, _M0 // _B0, _B0,
        [rk16, rc16, p['W1c'].reshape(K * 16, 16), row(p['b1c']), p['W1b'],
         row(p['b1b']), p['W1sc'], p['W2a'], row(p['b2a'])],
        [infl0, gh1, x], [64, 32])

    gh2 = _sc_gather(h2a, ep).reshape(_M1, 512)
    pmax = jnp.max(_sc_gather(skip0, ep).reshape(_M1, H, 64), axis=1)
    x1, h3a = _call(
        _t3_body, _M1 // _B1, _B1,
        [selc, tile4, rh, kpc1, rk32, rc32,
         p['W2c'].reshape(K * 32, 32), row(p['b2c']), p['W2b'], row(p['b2b']),
         p['W2sc'], p['W3a'], row(p['b3a'])],
        [gpp, q4_1, gh2, pmax], [128, 32])

    g3 = _sc_gather(h3a, e1).reshape(_M1, 512)
    x1b, h4a, infl1 = _call(
        _t4_body, _M1 // _B1, _B1,
        [selc, tile4, rh, kpc1, rk32, rc32,
         p['W3c'].reshape(K * 32, 32), row(p['b3c']), p['W3b'], row(p['b3b']),
         p['W4a'], row(p['b4a'])],
        [gp1, q4_1, g3, x1], [128, 32, _HK])

    g4 = _sc_gather(h4a, e1).reshape(_M1, 512)
    (x1f,) = _call(
        _t5_body, _M1 // _B1, _B1,
        [rk32, rc32, p['W4c'].reshape(K * 32, 32), row(p['b4c']), p['W4b'],
         row(p['b4b'])],
        [infl1, g4, x1b], [128])

    upidx = jnp.pad(upsamples_0[:, 0], (0, _M0 - N0))
    up = _sc_gather(x1f, upidx)[:_M0]
    (feat,) = _call(
        _tfeat_body, _M0 // _B0, _B0,
        [p['Wl'][:128], p['Wl'][128:], row(p['bl'])],
        [up, skip0], [32])
    mf = jnp.mean(_sc_gather(feat, e0).reshape(_M0, H, 32), axis=1)

    desc, scores = _call(
        _t6_body, _M0 // _B0, _B0,
        [p['Wl'][:128], p['Wl'][128:], row(p['bl'])],
        [up, skip0, mf], [32, 1])

    return desc[:N0], scores[:N0]
